# baseline (speedup 1.0000x reference)
import functools
import math

import jax
import jax.numpy as jnp
from jax import lax
from jax.experimental import pallas as pl
from jax.experimental.pallas import tpu as pltpu
from jax.experimental.pallas import tpu_sc as plsc

D_MODEL = 1024
VOCAB = 100000
SCALE = math.sqrt(D_MODEL)

NC = 2
NS = 16
LANES = 16
NW = NC * NS

ROWS = 4
COLS = 4096
B = ROWS * COLS
B_PER_W = B // NW
W_PER_ROW = COLS // B_PER_W
CHUNK = 8
NCHUNK = B_PER_W // CHUNK
NBUF = 4
NROUND = NCHUNK // NBUF


@functools.partial(
    pl.kernel,
    out_type=jax.ShapeDtypeStruct((B, D_MODEL), jnp.float32),
    mesh=plsc.VectorSubcoreMesh(core_axis_name="c", subcore_axis_name="s"),
    scratch_types=[
        pltpu.VMEM((B_PER_W,), jnp.int32),
        [pltpu.VMEM((CHUNK, D_MODEL), jnp.float32) for _ in range(NBUF)],
        [pltpu.VMEM((CHUNK, D_MODEL), jnp.float32) for _ in range(NBUF)],
        [pltpu.SemaphoreType.DMA for _ in range(NBUF)],
        [pltpu.SemaphoreType.DMA for _ in range(NBUF)],
    ],
)
def _emb_kernel(x_hbm, lut_hbm, out_hbm, idx_v, inb, outb, gsem, ssem):
    wid = lax.axis_index("s") * NC + lax.axis_index("c")
    base = wid * B_PER_W

    pltpu.sync_copy(
        x_hbm.at[wid // W_PER_ROW, pl.ds((wid % W_PER_ROW) * B_PER_W, B_PER_W)],
        idx_v)

    def gather(i, b):
        return pltpu.async_copy(
            lut_hbm.at[idx_v.at[pl.ds(i * CHUNK, CHUNK)]], inb[b], gsem[b])

    def wait_gather(b):
        pltpu.make_async_copy(
            lut_hbm.at[idx_v.at[pl.ds(0, CHUNK)]], inb[b], gsem[b]).wait()

    def scatter(i, b):
        return pltpu.async_copy(
            outb[b], out_hbm.at[pl.ds(base + i * CHUNK, CHUNK)], ssem[b])

    def wait_scatter(b):
        pltpu.make_async_copy(
            outb[b], out_hbm.at[pl.ds(base, CHUNK)], ssem[b]).wait()

    def scale(b):
        @plsc.parallel_loop(0, CHUNK, step=1, unroll=2)
        def row_body(r):
            for j in range(D_MODEL // LANES):
                s = pl.ds(j * LANES, LANES)
                outb[b][r, s] = inb[b][r, s] * SCALE

    for b in range(NBUF):
        gather(b, b)

    for b in range(NBUF):
        wait_gather(b)
        scale(b)
        scatter(b, b)
        gather(NBUF + b, b)

    def round_body(g, carry):
        i0 = g * NBUF
        for b in range(NBUF):
            wait_gather(b)
            wait_scatter(b)
            scale(b)
            gather(i0 + NBUF + b, b)
            scatter(i0 + b, b)
        return carry
    lax.fori_loop(1, NROUND - 1, round_body, 0)

    i0 = (NROUND - 1) * NBUF
    for b in range(NBUF):
        wait_gather(b)
        wait_scatter(b)
        scale(b)
        scatter(i0 + b, b)

    for b in range(NBUF):
        wait_scatter(b)


def kernel(x, lut):
    out = _emb_kernel(x.astype(jnp.int32), lut)
    return jnp.reshape(out, (x.shape[0], x.shape[1], D_MODEL))


# speedup vs baseline: 1.0257x; 1.0257x over previous
import functools
import math

import jax
import jax.numpy as jnp
from jax import lax
from jax.experimental import pallas as pl
from jax.experimental.pallas import tpu as pltpu
from jax.experimental.pallas import tpu_sc as plsc

D_MODEL = 1024
VOCAB = 100000
SCALE = math.sqrt(D_MODEL)

NC = 2
NS = 16
LANES = 16
NW = NC * NS

ROWS = 4
COLS = 4096
B = ROWS * COLS
B_PER_W = B // NW
W_PER_ROW = COLS // B_PER_W
CHUNK = 8
NCHUNK = B_PER_W // CHUNK
NBUF = 4
NROUND = NCHUNK // NBUF


@functools.partial(
    pl.kernel,
    out_type=jax.ShapeDtypeStruct((B, D_MODEL), jnp.float32),
    mesh=plsc.VectorSubcoreMesh(core_axis_name="c", subcore_axis_name="s"),
    scratch_types=[
        pltpu.VMEM((B_PER_W,), jnp.int32),
        [pltpu.VMEM((CHUNK, D_MODEL), jnp.float32) for _ in range(NBUF)],
        [pltpu.VMEM((CHUNK, D_MODEL), jnp.float32) for _ in range(NBUF)],
        [pltpu.SemaphoreType.DMA for _ in range(NBUF)],
        [pltpu.SemaphoreType.DMA for _ in range(NBUF)],
    ],
)
def _emb_kernel(x_hbm, lut_hbm, out_hbm, idx_v, inb, outb, gsem, ssem):
    wid = lax.axis_index("s") * NC + lax.axis_index("c")
    base = wid * B_PER_W

    pltpu.sync_copy(
        x_hbm.at[wid // W_PER_ROW, pl.ds((wid % W_PER_ROW) * B_PER_W, B_PER_W)],
        idx_v)

    def gather(i, b):
        return pltpu.async_copy(
            lut_hbm.at[idx_v.at[pl.ds(i * CHUNK, CHUNK)]], inb[b], gsem[b])

    def wait_gather(b):
        pltpu.make_async_copy(
            lut_hbm.at[idx_v.at[pl.ds(0, CHUNK)]], inb[b], gsem[b]).wait()

    def scatter(i, b):
        return pltpu.async_copy(
            outb[b], out_hbm.at[pl.ds(base + i * CHUNK, CHUNK)], ssem[b])

    def wait_scatter(b):
        pltpu.make_async_copy(
            outb[b], out_hbm.at[pl.ds(base, CHUNK)], ssem[b]).wait()

    def scale(b):
        def row_body(r, carry):
            for j in range(D_MODEL // LANES):
                s = pl.ds(j * LANES, LANES)
                outb[b][r, s] = inb[b][r, s] * SCALE
            return carry
        lax.fori_loop(0, CHUNK, row_body, 0)

    for b in range(NBUF):
        gather(b, b)

    for b in range(NBUF):
        wait_gather(b)
        scale(b)
        scatter(b, b)
        gather(NBUF + b, b)

    def round_body(g, carry):
        i0 = g * NBUF
        for b in range(NBUF):
            wait_gather(b)
            wait_scatter(b)
            scale(b)
            gather(i0 + NBUF + b, b)
            scatter(i0 + b, b)
        return carry
    lax.fori_loop(1, NROUND - 1, round_body, 0)

    i0 = (NROUND - 1) * NBUF
    for b in range(NBUF):
        wait_gather(b)
        wait_scatter(b)
        scale(b)
        scatter(i0 + b, b)

    for b in range(NBUF):
        wait_scatter(b)


def kernel(x, lut):
    out = _emb_kernel(x.astype(jnp.int32), lut)
    return jnp.reshape(out, (x.shape[0], x.shape[1], D_MODEL))
